# baseline (speedup 1.0000x reference)
import functools

import jax
import jax.numpy as jnp
from jax import lax
from jax.experimental import pallas as pl
from jax.experimental.pallas import tpu as pltpu
from jax.experimental.pallas import tpu_sc as plsc

OIM_SCALAR = 30.0
_BLOCK_ROWS = 8000


def _oim_tc_body(lut_ref, x_ref, cls_ref, mask_ref, g_ref, out_ref,
                 m_ref, s_ref):
    j = pl.program_id(0)

    @pl.when(j == 0)
    def _init():
        m_ref[...] = jnp.full(m_ref.shape, -jnp.inf, dtype=jnp.float32)
        s_ref[...] = jnp.zeros(s_ref.shape, dtype=jnp.float32)

    scale = cls_ref[...] * OIM_SCALAR
    val = lax.dot_general(
        lut_ref[...], x_ref[...],
        (((1,), (1,)), ((), ())),
        preferred_element_type=jnp.float32,
        precision=lax.Precision.HIGHEST,
    )
    val = val * scale
    bm = jnp.max(val, axis=0, keepdims=True)
    m_old = m_ref[...]
    m_new = jnp.maximum(m_old, bm)
    s_ref[...] = (s_ref[...] * jnp.exp(m_old - m_new)
                  + jnp.sum(jnp.exp(val - m_new), axis=0, keepdims=True))
    m_ref[...] = m_new

    @pl.when(j == pl.num_programs(0) - 1)
    def _finish():
        lse = m_ref[...] + jnp.log(s_ref[...])
        mask = mask_ref[...]
        w = mask * scale
        r = jnp.sum(x_ref[...] * g_ref[...], axis=1, keepdims=True)
        picked_sum = lax.dot_general(
            w, r, (((1,), (0,)), ((), ())),
            preferred_element_type=jnp.float32,
            precision=lax.Precision.HIGHEST,
        )
        num = jnp.sum(mask * lse) - picked_sum[0, 0]
        den = jnp.sum(mask)
        out_ref[0, 0] = num / den


def _make_sc_gather(num_pids, feat, batch):
    info = plsc.get_sparse_core_info()
    nc = info.num_cores
    rows_per_worker = 16
    n_workers = batch // rows_per_worker
    mesh = plsc.VectorSubcoreMesh(core_axis_name="c", subcore_axis_name="s")

    @functools.partial(
        pl.kernel,
        mesh=mesh,
        out_type=jax.ShapeDtypeStruct((batch, feat), jnp.float32),
        scratch_types=[
            pltpu.VMEM((rows_per_worker,), jnp.int32),
            pltpu.VMEM((rows_per_worker, feat), jnp.float32),
            pltpu.SemaphoreType.DMA,
        ],
    )
    def gather_k(tgt_hbm, lut_hbm, out_hbm, idx_v, rows_v, sem):
        wid = lax.axis_index("s") * nc + lax.axis_index("c")

        @pl.when(wid < n_workers)
        def _():
            base = wid * rows_per_worker
            pltpu.sync_copy(tgt_hbm.at[pl.ds(base, rows_per_worker)], idx_v)
            idx_v[...] = jnp.maximum(idx_v[...] - 1, 0)
            pltpu.async_copy(lut_hbm.at[idx_v], rows_v, sem).wait()
            pltpu.sync_copy(rows_v, out_hbm.at[pl.ds(base, rows_per_worker)])

    return gather_k


def kernel(inputs, roi_label, cls_scores, fidelity, lut):
    del fidelity
    batch, feat = inputs.shape
    num_pids = lut.shape[0]
    targets = roi_label.reshape(-1).astype(jnp.int32)

    g = _make_sc_gather(num_pids, feat, batch)(targets, lut)

    cls_row = cls_scores.reshape(1, batch)
    mask_row = (targets > 0).astype(jnp.float32).reshape(1, batch)

    block_rows = _BLOCK_ROWS
    grid = num_pids // block_rows

    out = pl.pallas_call(
        _oim_tc_body,
        grid=(grid,),
        in_specs=[
            pl.BlockSpec((block_rows, feat), lambda j: (j, 0)),
            pl.BlockSpec((batch, feat), lambda j: (0, 0)),
            pl.BlockSpec((1, batch), lambda j: (0, 0)),
            pl.BlockSpec((1, batch), lambda j: (0, 0)),
            pl.BlockSpec((batch, feat), lambda j: (0, 0)),
        ],
        out_specs=pl.BlockSpec(memory_space=pltpu.SMEM),
        out_shape=jax.ShapeDtypeStruct((1, 1), jnp.float32),
        scratch_shapes=[
            pltpu.VMEM((1, batch), jnp.float32),
            pltpu.VMEM((1, batch), jnp.float32),
        ],
        compiler_params=pltpu.CompilerParams(
            dimension_semantics=("arbitrary",),
        ),
    )(lut, inputs, cls_row, mask_row, g)

    return out[0, 0]


# speedup vs baseline: 2.1242x; 2.1242x over previous
import functools

import jax
import jax.numpy as jnp
from jax import lax
from jax.experimental import pallas as pl
from jax.experimental.pallas import tpu as pltpu
from jax.experimental.pallas import tpu_sc as plsc

OIM_SCALAR = 30.0
_BLOCK_ROWS = 8000


def _oim_tc_body(lut_ref, xs_ref, maskr_ref, maskc_ref, g_ref, out_ref,
                 m_ref, s_ref):
    j = pl.program_id(0)

    @pl.when(j == 0)
    def _init():
        m_ref[...] = jnp.full(m_ref.shape, -jnp.inf, dtype=jnp.float32)
        s_ref[...] = jnp.zeros(s_ref.shape, dtype=jnp.float32)

    val = lax.dot_general(
        lut_ref[...], xs_ref[...],
        (((1,), (1,)), ((), ())),
        preferred_element_type=jnp.float32,
    )
    nb, b = val.shape
    val3 = val.reshape(nb // 8, 8, b)
    bm = jnp.max(val3, axis=0)
    m_old = m_ref[...]
    m_new = jnp.maximum(m_old, bm)
    s_ref[...] = (s_ref[...] * jnp.exp(m_old - m_new)
                  + jnp.sum(jnp.exp(val3 - m_new[None]), axis=0))
    m_ref[...] = m_new

    @pl.when(j == pl.num_programs(0) - 1)
    def _finish():
        m_all = m_ref[...]
        m_fin = jnp.max(m_all, axis=0, keepdims=True)
        s_fin = jnp.sum(s_ref[...] * jnp.exp(m_all - m_fin),
                        axis=0, keepdims=True)
        lse = m_fin + jnp.log(s_fin)
        picked = jnp.sum(xs_ref[...] * g_ref[...], axis=1,
                         keepdims=True)
        num = (jnp.sum(maskr_ref[...] * lse)
               - jnp.sum(maskc_ref[...] * picked))
        den = jnp.sum(maskr_ref[...])
        out_ref[0, 0] = num / den


def _make_sc_gather(feat, batch):
    info = plsc.get_sparse_core_info()
    nc = info.num_cores
    rows_per_worker = 16
    n_workers = batch // rows_per_worker
    mesh = plsc.VectorSubcoreMesh(core_axis_name="c", subcore_axis_name="s")

    @functools.partial(
        pl.kernel,
        mesh=mesh,
        out_type=jax.ShapeDtypeStruct((batch, feat), jnp.float32),
        scratch_types=[
            pltpu.VMEM((rows_per_worker,), jnp.int32),
            pltpu.VMEM((rows_per_worker, feat), jnp.float32),
            pltpu.SemaphoreType.DMA,
        ],
    )
    def gather_k(tgt_hbm, lut_hbm, out_hbm, idx_v, rows_v, sem):
        wid = lax.axis_index("s") * nc + lax.axis_index("c")

        @pl.when(wid < n_workers)
        def _():
            base = wid * rows_per_worker
            pltpu.sync_copy(tgt_hbm.at[pl.ds(base, rows_per_worker)], idx_v)
            idx_v[...] = jnp.maximum(idx_v[...] - 1, 0)
            pltpu.async_copy(lut_hbm.at[idx_v], rows_v, sem).wait()
            pltpu.sync_copy(rows_v, out_hbm.at[pl.ds(base, rows_per_worker)])

    return gather_k


def kernel(inputs, roi_label, cls_scores, fidelity, lut):
    del fidelity
    batch, feat = inputs.shape
    num_pids = lut.shape[0]
    targets = roi_label.reshape(-1).astype(jnp.int32)

    g = _make_sc_gather(feat, batch)(targets, lut)

    xs = inputs * (OIM_SCALAR * cls_scores)[:, None]
    maskf = (targets > 0).astype(jnp.float32)
    mask_row = maskf.reshape(1, batch)
    mask_col = maskf.reshape(batch, 1)

    block_rows = _BLOCK_ROWS
    grid = num_pids // block_rows

    out = pl.pallas_call(
        _oim_tc_body,
        grid=(grid,),
        in_specs=[
            pl.BlockSpec((block_rows, feat), lambda j: (j, 0)),
            pl.BlockSpec((batch, feat), lambda j: (0, 0)),
            pl.BlockSpec((1, batch), lambda j: (0, 0)),
            pl.BlockSpec((batch, 1), lambda j: (0, 0)),
            pl.BlockSpec((batch, feat), lambda j: (0, 0)),
        ],
        out_specs=pl.BlockSpec(memory_space=pltpu.SMEM),
        out_shape=jax.ShapeDtypeStruct((1, 1), jnp.float32),
        scratch_shapes=[
            pltpu.VMEM((8, batch), jnp.float32),
            pltpu.VMEM((8, batch), jnp.float32),
        ],
        compiler_params=pltpu.CompilerParams(
            dimension_semantics=("arbitrary",),
        ),
    )(lut, xs, mask_row, mask_col, g)

    return out[0, 0]


# speedup vs baseline: 2.3730x; 1.1171x over previous
import functools
import math

import jax
import jax.numpy as jnp
from jax import lax
from jax.experimental import pallas as pl
from jax.experimental.pallas import tpu as pltpu
from jax.experimental.pallas import tpu_sc as plsc

OIM_SCALAR = 30.0
_LN2 = math.log(2.0)
_LOG2E = 1.0 / _LN2
_BLOCK_ROWS = 8000
_CHUNKS = 4
_ACCW = 16


def _oim_tc_body(lut_ref, xs_ref, maskr_ref, maskc_ref, g_ref, out_ref,
                 m_ref, s_ref):
    j = pl.program_id(0)

    @pl.when(j == 0)
    def _init():
        m_ref[...] = jnp.full(m_ref.shape, -jnp.inf, dtype=jnp.float32)
        s_ref[...] = jnp.zeros(s_ref.shape, dtype=jnp.float32)

    nb = lut_ref.shape[0]
    b = xs_ref.shape[0]
    rows = nb // _CHUNKS
    for c in range(_CHUNKS):
        val = lax.dot_general(
            lut_ref[pl.ds(c * rows, rows), :], xs_ref[...],
            (((1,), (1,)), ((), ())),
            preferred_element_type=jnp.float32,
        )
        val3 = val.reshape(rows // _ACCW, _ACCW, b)
        bm = jnp.max(val3, axis=0)
        a = pl.ds(c * _ACCW, _ACCW)
        m_old = m_ref[a, :]
        m_new = jnp.maximum(m_old, bm)
        s_ref[a, :] = (s_ref[a, :] * jnp.exp2(m_old - m_new)
                       + jnp.sum(jnp.exp2(val3 - m_new[None]), axis=0))
        m_ref[a, :] = m_new

    @pl.when(j == pl.num_programs(0) - 1)
    def _finish():
        m_all = m_ref[...]
        m_fin = jnp.max(m_all, axis=0, keepdims=True)
        s_fin = jnp.sum(s_ref[...] * jnp.exp2(m_all - m_fin),
                        axis=0, keepdims=True)
        lse = _LN2 * m_fin + jnp.log(s_fin)
        picked = jnp.sum(xs_ref[...] * g_ref[...], axis=1,
                         keepdims=True)
        num = (jnp.sum(maskr_ref[...] * lse)
               - _LN2 * jnp.sum(maskc_ref[...] * picked))
        den = jnp.sum(maskr_ref[...])
        out_ref[0, 0] = num / den


def _make_sc_gather(feat, batch):
    info = plsc.get_sparse_core_info()
    nc = info.num_cores
    rows_per_worker = 16
    n_workers = batch // rows_per_worker
    mesh = plsc.VectorSubcoreMesh(core_axis_name="c", subcore_axis_name="s")

    @functools.partial(
        pl.kernel,
        mesh=mesh,
        out_type=jax.ShapeDtypeStruct((batch, feat), jnp.float32),
        scratch_types=[
            pltpu.VMEM((rows_per_worker,), jnp.int32),
            pltpu.VMEM((rows_per_worker, feat), jnp.float32),
            pltpu.SemaphoreType.DMA,
        ],
    )
    def gather_k(tgt_hbm, lut_hbm, out_hbm, idx_v, rows_v, sem):
        wid = lax.axis_index("s") * nc + lax.axis_index("c")

        @pl.when(wid < n_workers)
        def _():
            base = wid * rows_per_worker
            pltpu.sync_copy(tgt_hbm.at[pl.ds(base, rows_per_worker)], idx_v)
            idx_v[...] = jnp.maximum(idx_v[...] - 1, 0)
            pltpu.async_copy(lut_hbm.at[idx_v], rows_v, sem).wait()
            pltpu.sync_copy(rows_v, out_hbm.at[pl.ds(base, rows_per_worker)])

    return gather_k


def kernel(inputs, roi_label, cls_scores, fidelity, lut):
    del fidelity
    batch, feat = inputs.shape
    num_pids = lut.shape[0]
    targets = roi_label.reshape(-1).astype(jnp.int32)

    g = _make_sc_gather(feat, batch)(targets, lut)

    xs = inputs * (_LOG2E * OIM_SCALAR * cls_scores)[:, None]
    maskf = (targets > 0).astype(jnp.float32)
    mask_row = maskf.reshape(1, batch)
    mask_col = maskf.reshape(batch, 1)

    block_rows = _BLOCK_ROWS
    grid = num_pids // block_rows

    out = pl.pallas_call(
        _oim_tc_body,
        grid=(grid,),
        in_specs=[
            pl.BlockSpec((block_rows, feat), lambda j: (j, 0)),
            pl.BlockSpec((batch, feat), lambda j: (0, 0)),
            pl.BlockSpec((1, batch), lambda j: (0, 0)),
            pl.BlockSpec((batch, 1), lambda j: (0, 0)),
            pl.BlockSpec((batch, feat), lambda j: (0, 0)),
        ],
        out_specs=pl.BlockSpec(memory_space=pltpu.SMEM),
        out_shape=jax.ShapeDtypeStruct((1, 1), jnp.float32),
        scratch_shapes=[
            pltpu.VMEM((_CHUNKS * _ACCW, batch), jnp.float32),
            pltpu.VMEM((_CHUNKS * _ACCW, batch), jnp.float32),
        ],
        compiler_params=pltpu.CompilerParams(
            dimension_semantics=("arbitrary",),
        ),
    )(lut, xs, mask_row, mask_col, g)

    return out[0, 0]


# speedup vs baseline: 2.5658x; 1.0812x over previous
import functools
import math

import jax
import jax.numpy as jnp
from jax import lax
from jax.experimental import pallas as pl
from jax.experimental.pallas import tpu as pltpu
from jax.experimental.pallas import tpu_sc as plsc

OIM_SCALAR = 30.0
_LN2 = math.log(2.0)
_LOG2E = 1.0 / _LN2
_BLOCK_ROWS = 10000
_CHUNKS = 5
_ACCW = 16


def _oim_tc_body(lut_ref, xs_ref, maskr_ref, maskc_ref, g_ref, out_ref,
                 m_ref, s_ref):
    j = pl.program_id(0)

    @pl.when(j == 0)
    def _init():
        m_ref[...] = jnp.full(m_ref.shape, -jnp.inf, dtype=jnp.float32)
        s_ref[...] = jnp.zeros(s_ref.shape, dtype=jnp.float32)

    nb = lut_ref.shape[0]
    b = xs_ref.shape[0]
    rows = nb // _CHUNKS
    for c in range(_CHUNKS):
        val = lax.dot_general(
            lut_ref[pl.ds(c * rows, rows), :], xs_ref[...],
            (((1,), (1,)), ((), ())),
            preferred_element_type=jnp.float32,
        )
        val3 = val.reshape(rows // _ACCW, _ACCW, b)
        bm = jnp.max(val3, axis=0)
        a = pl.ds(c * _ACCW, _ACCW)
        m_old = m_ref[a, :]
        m_new = jnp.maximum(m_old, bm)
        s_ref[a, :] = (s_ref[a, :] * jnp.exp2(m_old - m_new)
                       + jnp.sum(jnp.exp2(val3 - m_new[None]), axis=0))
        m_ref[a, :] = m_new

    @pl.when(j == pl.num_programs(0) - 1)
    def _finish():
        m_all = m_ref[...]
        m_fin = jnp.max(m_all, axis=0, keepdims=True)
        s_fin = jnp.sum(s_ref[...] * jnp.exp2(m_all - m_fin),
                        axis=0, keepdims=True)
        lse = _LN2 * m_fin + jnp.log(s_fin)
        picked = jnp.sum(xs_ref[...] * g_ref[...], axis=1,
                         keepdims=True)
        num = (jnp.sum(maskr_ref[...] * lse)
               - _LN2 * jnp.sum(maskc_ref[...] * picked))
        den = jnp.sum(maskr_ref[...])
        out_ref[0, 0] = num / den


def _make_sc_gather(feat, batch):
    info = plsc.get_sparse_core_info()
    nc = info.num_cores
    rows_per_worker = 16
    n_workers = batch // rows_per_worker
    mesh = plsc.VectorSubcoreMesh(core_axis_name="c", subcore_axis_name="s")

    @functools.partial(
        pl.kernel,
        mesh=mesh,
        out_type=jax.ShapeDtypeStruct((batch, feat), jnp.float32),
        scratch_types=[
            pltpu.VMEM((rows_per_worker,), jnp.int32),
            pltpu.VMEM((rows_per_worker, feat), jnp.float32),
            pltpu.SemaphoreType.DMA,
        ],
    )
    def gather_k(tgt_hbm, lut_hbm, out_hbm, idx_v, rows_v, sem):
        wid = lax.axis_index("s") * nc + lax.axis_index("c")

        @pl.when(wid < n_workers)
        def _():
            base = wid * rows_per_worker
            pltpu.sync_copy(tgt_hbm.at[pl.ds(base, rows_per_worker)], idx_v)
            idx_v[...] = jnp.maximum(idx_v[...] - 1, 0)
            pltpu.async_copy(lut_hbm.at[idx_v], rows_v, sem).wait()
            pltpu.sync_copy(rows_v, out_hbm.at[pl.ds(base, rows_per_worker)])

    return gather_k


def kernel(inputs, roi_label, cls_scores, fidelity, lut):
    del fidelity
    batch, feat = inputs.shape
    num_pids = lut.shape[0]
    targets = roi_label.reshape(-1).astype(jnp.int32)

    g = _make_sc_gather(feat, batch)(targets, lut)

    xs = inputs * (_LOG2E * OIM_SCALAR * cls_scores)[:, None]
    maskf = (targets > 0).astype(jnp.float32)
    mask_row = maskf.reshape(1, batch)
    mask_col = maskf.reshape(batch, 1)

    block_rows = _BLOCK_ROWS
    grid = num_pids // block_rows

    out = pl.pallas_call(
        _oim_tc_body,
        grid=(grid,),
        in_specs=[
            pl.BlockSpec((block_rows, feat), lambda j: (j, 0)),
            pl.BlockSpec((batch, feat), lambda j: (0, 0)),
            pl.BlockSpec((1, batch), lambda j: (0, 0)),
            pl.BlockSpec((batch, 1), lambda j: (0, 0)),
            pl.BlockSpec((batch, feat), lambda j: (0, 0)),
        ],
        out_specs=pl.BlockSpec(memory_space=pltpu.SMEM),
        out_shape=jax.ShapeDtypeStruct((1, 1), jnp.float32),
        scratch_shapes=[
            pltpu.VMEM((_CHUNKS * _ACCW, batch), jnp.float32),
            pltpu.VMEM((_CHUNKS * _ACCW, batch), jnp.float32),
        ],
        compiler_params=pltpu.CompilerParams(
            dimension_semantics=("arbitrary",),
        ),
    )(lut, xs, mask_row, mask_col, g)

    return out[0, 0]


# speedup vs baseline: 2.8907x; 1.1267x over previous
import functools
import math

import jax
import jax.numpy as jnp
from jax import lax
from jax.experimental import pallas as pl
from jax.experimental.pallas import tpu as pltpu
from jax.experimental.pallas import tpu_sc as plsc

OIM_SCALAR = 30.0
_LN2 = math.log(2.0)
_LOG2E = 1.0 / _LN2
_BLOCK_ROWS = 20000
_CHUNKS = 10
_ACCW = 16


def _oim_tc_body(lut_ref, xs_ref, maskr_ref, maskc_ref, g_ref, out_ref,
                 m_ref, s_ref):
    j = pl.program_id(0)

    @pl.when(j == 0)
    def _init():
        m_ref[...] = jnp.full(m_ref.shape, -jnp.inf, dtype=jnp.float32)
        s_ref[...] = jnp.zeros(s_ref.shape, dtype=jnp.float32)

    nb = lut_ref.shape[0]
    b = xs_ref.shape[0]
    rows = nb // _CHUNKS
    for c in range(_CHUNKS):
        val = lax.dot_general(
            lut_ref[pl.ds(c * rows, rows), :], xs_ref[...],
            (((1,), (1,)), ((), ())),
            preferred_element_type=jnp.float32,
        )
        val3 = val.reshape(rows // _ACCW, _ACCW, b)
        bm = jnp.max(val3, axis=0)
        a = pl.ds(c * _ACCW, _ACCW)
        m_old = m_ref[a, :]
        m_new = jnp.maximum(m_old, bm)
        s_ref[a, :] = (s_ref[a, :] * jnp.exp2(m_old - m_new)
                       + jnp.sum(jnp.exp2(val3 - m_new[None]), axis=0))
        m_ref[a, :] = m_new

    @pl.when(j == pl.num_programs(0) - 1)
    def _finish():
        m_all = m_ref[...]
        m_fin = jnp.max(m_all, axis=0, keepdims=True)
        s_fin = jnp.sum(s_ref[...] * jnp.exp2(m_all - m_fin),
                        axis=0, keepdims=True)
        lse = _LN2 * m_fin + jnp.log(s_fin)
        picked = jnp.sum(xs_ref[...] * g_ref[...], axis=1,
                         keepdims=True)
        num = (jnp.sum(maskr_ref[...] * lse)
               - _LN2 * jnp.sum(maskc_ref[...] * picked))
        den = jnp.sum(maskr_ref[...])
        out_ref[0, 0] = num / den


def _make_sc_gather(feat, batch):
    info = plsc.get_sparse_core_info()
    nc = info.num_cores
    rows_per_worker = 16
    n_workers = batch // rows_per_worker
    mesh = plsc.VectorSubcoreMesh(core_axis_name="c", subcore_axis_name="s")

    @functools.partial(
        pl.kernel,
        mesh=mesh,
        out_type=jax.ShapeDtypeStruct((batch, feat), jnp.float32),
        scratch_types=[
            pltpu.VMEM((rows_per_worker,), jnp.int32),
            pltpu.VMEM((rows_per_worker, feat), jnp.float32),
            pltpu.SemaphoreType.DMA,
        ],
    )
    def gather_k(tgt_hbm, lut_hbm, out_hbm, idx_v, rows_v, sem):
        wid = lax.axis_index("s") * nc + lax.axis_index("c")

        @pl.when(wid < n_workers)
        def _():
            base = wid * rows_per_worker
            pltpu.sync_copy(tgt_hbm.at[pl.ds(base, rows_per_worker)], idx_v)
            idx_v[...] = jnp.maximum(idx_v[...] - 1, 0)
            pltpu.async_copy(lut_hbm.at[idx_v], rows_v, sem).wait()
            pltpu.sync_copy(rows_v, out_hbm.at[pl.ds(base, rows_per_worker)])

    return gather_k


def kernel(inputs, roi_label, cls_scores, fidelity, lut):
    del fidelity
    batch, feat = inputs.shape
    num_pids = lut.shape[0]
    targets = roi_label.reshape(-1).astype(jnp.int32)

    g = _make_sc_gather(feat, batch)(targets, lut)

    xs = inputs * (_LOG2E * OIM_SCALAR * cls_scores)[:, None]
    maskf = (targets > 0).astype(jnp.float32)
    mask_row = maskf.reshape(1, batch)
    mask_col = maskf.reshape(batch, 1)

    block_rows = _BLOCK_ROWS
    grid = num_pids // block_rows

    out = pl.pallas_call(
        _oim_tc_body,
        grid=(grid,),
        in_specs=[
            pl.BlockSpec((block_rows, feat), lambda j: (j, 0)),
            pl.BlockSpec((batch, feat), lambda j: (0, 0)),
            pl.BlockSpec((1, batch), lambda j: (0, 0)),
            pl.BlockSpec((batch, 1), lambda j: (0, 0)),
            pl.BlockSpec((batch, feat), lambda j: (0, 0)),
        ],
        out_specs=pl.BlockSpec(memory_space=pltpu.SMEM),
        out_shape=jax.ShapeDtypeStruct((1, 1), jnp.float32),
        scratch_shapes=[
            pltpu.VMEM((_CHUNKS * _ACCW, batch), jnp.float32),
            pltpu.VMEM((_CHUNKS * _ACCW, batch), jnp.float32),
        ],
        compiler_params=pltpu.CompilerParams(
            dimension_semantics=("arbitrary",),
        ),
    )(lut, xs, mask_row, mask_col, g)

    return out[0, 0]


# speedup vs baseline: 3.0541x; 1.0565x over previous
import functools
import math

import jax
import jax.numpy as jnp
from jax import lax
from jax.experimental import pallas as pl
from jax.experimental.pallas import tpu as pltpu
from jax.experimental.pallas import tpu_sc as plsc

OIM_SCALAR = 30.0
_LN2 = math.log(2.0)
_LOG2E = 1.0 / _LN2
_BLOCK_ROWS = 40000
_CHUNKS = 20
_ACCW = 16


def _oim_tc_body(lut_ref, xs_ref, maskr_ref, maskc_ref, g_ref, out_ref,
                 m_ref, s_ref):
    j = pl.program_id(0)

    @pl.when(j == 0)
    def _init():
        m_ref[...] = jnp.full(m_ref.shape, -jnp.inf, dtype=jnp.float32)
        s_ref[...] = jnp.zeros(s_ref.shape, dtype=jnp.float32)

    nb = lut_ref.shape[0]
    b = xs_ref.shape[0]
    rows = nb // _CHUNKS
    for c in range(_CHUNKS):
        val = lax.dot_general(
            lut_ref[pl.ds(c * rows, rows), :], xs_ref[...],
            (((1,), (1,)), ((), ())),
            preferred_element_type=jnp.float32,
        )
        val3 = val.reshape(rows // _ACCW, _ACCW, b)
        bm = jnp.max(val3, axis=0)
        a = pl.ds(c * _ACCW, _ACCW)
        m_old = m_ref[a, :]
        m_new = jnp.maximum(m_old, bm)
        s_ref[a, :] = (s_ref[a, :] * jnp.exp2(m_old - m_new)
                       + jnp.sum(jnp.exp2(val3 - m_new[None]), axis=0))
        m_ref[a, :] = m_new

    @pl.when(j == pl.num_programs(0) - 1)
    def _finish():
        m_all = m_ref[...]
        m_fin = jnp.max(m_all, axis=0, keepdims=True)
        s_fin = jnp.sum(s_ref[...] * jnp.exp2(m_all - m_fin),
                        axis=0, keepdims=True)
        lse = _LN2 * m_fin + jnp.log(s_fin)
        picked = jnp.sum(xs_ref[...] * g_ref[...], axis=1,
                         keepdims=True)
        num = (jnp.sum(maskr_ref[...] * lse)
               - _LN2 * jnp.sum(maskc_ref[...] * picked))
        den = jnp.sum(maskr_ref[...])
        out_ref[0, 0] = num / den


def _make_sc_gather(feat, batch):
    info = plsc.get_sparse_core_info()
    nc = info.num_cores
    rows_per_worker = 16
    n_workers = batch // rows_per_worker
    mesh = plsc.VectorSubcoreMesh(core_axis_name="c", subcore_axis_name="s")

    @functools.partial(
        pl.kernel,
        mesh=mesh,
        out_type=jax.ShapeDtypeStruct((batch, feat), jnp.float32),
        scratch_types=[
            pltpu.VMEM((rows_per_worker,), jnp.int32),
            pltpu.VMEM((rows_per_worker, feat), jnp.float32),
            pltpu.SemaphoreType.DMA,
        ],
    )
    def gather_k(tgt_hbm, lut_hbm, out_hbm, idx_v, rows_v, sem):
        wid = lax.axis_index("s") * nc + lax.axis_index("c")

        @pl.when(wid < n_workers)
        def _():
            base = wid * rows_per_worker
            pltpu.sync_copy(tgt_hbm.at[pl.ds(base, rows_per_worker)], idx_v)
            idx_v[...] = jnp.maximum(idx_v[...] - 1, 0)
            pltpu.async_copy(lut_hbm.at[idx_v], rows_v, sem).wait()
            pltpu.sync_copy(rows_v, out_hbm.at[pl.ds(base, rows_per_worker)])

    return gather_k


def kernel(inputs, roi_label, cls_scores, fidelity, lut):
    del fidelity
    batch, feat = inputs.shape
    num_pids = lut.shape[0]
    targets = roi_label.reshape(-1).astype(jnp.int32)

    g = _make_sc_gather(feat, batch)(targets, lut)

    xs = inputs * (_LOG2E * OIM_SCALAR * cls_scores)[:, None]
    maskf = (targets > 0).astype(jnp.float32)
    mask_row = maskf.reshape(1, batch)
    mask_col = maskf.reshape(batch, 1)

    block_rows = _BLOCK_ROWS
    grid = num_pids // block_rows

    out = pl.pallas_call(
        _oim_tc_body,
        grid=(grid,),
        in_specs=[
            pl.BlockSpec((block_rows, feat), lambda j: (j, 0)),
            pl.BlockSpec((batch, feat), lambda j: (0, 0)),
            pl.BlockSpec((1, batch), lambda j: (0, 0)),
            pl.BlockSpec((batch, 1), lambda j: (0, 0)),
            pl.BlockSpec((batch, feat), lambda j: (0, 0)),
        ],
        out_specs=pl.BlockSpec(memory_space=pltpu.SMEM),
        out_shape=jax.ShapeDtypeStruct((1, 1), jnp.float32),
        scratch_shapes=[
            pltpu.VMEM((_CHUNKS * _ACCW, batch), jnp.float32),
            pltpu.VMEM((_CHUNKS * _ACCW, batch), jnp.float32),
        ],
        compiler_params=pltpu.CompilerParams(
            dimension_semantics=("arbitrary",),
        ),
    )(lut, xs, mask_row, mask_col, g)

    return out[0, 0]


# speedup vs baseline: 3.1163x; 1.0204x over previous
import functools
import math

import jax
import jax.numpy as jnp
from jax import lax
from jax.experimental import pallas as pl
from jax.experimental.pallas import tpu as pltpu
from jax.experimental.pallas import tpu_sc as plsc

OIM_SCALAR = 30.0
_LN2 = math.log(2.0)
_LOG2E = 1.0 / _LN2
_BLOCK_ROWS = 50000
_CHUNKS = 25
_ACCW = 16


def _oim_tc_body(lut_ref, xs_ref, maskr_ref, maskc_ref, g_ref, out_ref,
                 m_ref, s_ref):
    j = pl.program_id(0)

    @pl.when(j == 0)
    def _init():
        m_ref[...] = jnp.full(m_ref.shape, -jnp.inf, dtype=jnp.float32)
        s_ref[...] = jnp.zeros(s_ref.shape, dtype=jnp.float32)

    nb = lut_ref.shape[0]
    b = xs_ref.shape[0]
    rows = nb // _CHUNKS
    for c in range(_CHUNKS):
        val = lax.dot_general(
            lut_ref[pl.ds(c * rows, rows), :], xs_ref[...],
            (((1,), (1,)), ((), ())),
            preferred_element_type=jnp.float32,
        )
        val3 = val.reshape(rows // _ACCW, _ACCW, b)
        bm = jnp.max(val3, axis=0)
        a = pl.ds(c * _ACCW, _ACCW)
        m_old = m_ref[a, :]
        m_new = jnp.maximum(m_old, bm)
        s_ref[a, :] = (s_ref[a, :] * jnp.exp2(m_old - m_new)
                       + jnp.sum(jnp.exp2(val3 - m_new[None]), axis=0))
        m_ref[a, :] = m_new

    @pl.when(j == pl.num_programs(0) - 1)
    def _finish():
        m_all = m_ref[...]
        m_fin = jnp.max(m_all, axis=0, keepdims=True)
        s_fin = jnp.sum(s_ref[...] * jnp.exp2(m_all - m_fin),
                        axis=0, keepdims=True)
        lse = _LN2 * m_fin + jnp.log(s_fin)
        picked = jnp.sum(xs_ref[...] * g_ref[...], axis=1,
                         keepdims=True)
        num = (jnp.sum(maskr_ref[...] * lse)
               - _LN2 * jnp.sum(maskc_ref[...] * picked))
        den = jnp.sum(maskr_ref[...])
        out_ref[0, 0] = num / den


def _make_sc_gather(feat, batch):
    info = plsc.get_sparse_core_info()
    nc = info.num_cores
    rows_per_worker = 16
    n_workers = batch // rows_per_worker
    mesh = plsc.VectorSubcoreMesh(core_axis_name="c", subcore_axis_name="s")

    @functools.partial(
        pl.kernel,
        mesh=mesh,
        out_type=jax.ShapeDtypeStruct((batch, feat), jnp.float32),
        scratch_types=[
            pltpu.VMEM((rows_per_worker,), jnp.int32),
            pltpu.VMEM((rows_per_worker, feat), jnp.float32),
            pltpu.SemaphoreType.DMA,
        ],
    )
    def gather_k(tgt_hbm, lut_hbm, out_hbm, idx_v, rows_v, sem):
        wid = lax.axis_index("s") * nc + lax.axis_index("c")

        @pl.when(wid < n_workers)
        def _():
            base = wid * rows_per_worker
            pltpu.sync_copy(tgt_hbm.at[pl.ds(base, rows_per_worker)], idx_v)
            idx_v[...] = jnp.maximum(idx_v[...] - 1, 0)
            pltpu.async_copy(lut_hbm.at[idx_v], rows_v, sem).wait()
            pltpu.sync_copy(rows_v, out_hbm.at[pl.ds(base, rows_per_worker)])

    return gather_k


def kernel(inputs, roi_label, cls_scores, fidelity, lut):
    del fidelity
    batch, feat = inputs.shape
    num_pids = lut.shape[0]
    targets = roi_label.reshape(-1).astype(jnp.int32)

    g = _make_sc_gather(feat, batch)(targets, lut)

    xs = inputs * (_LOG2E * OIM_SCALAR * cls_scores)[:, None]
    maskf = (targets > 0).astype(jnp.float32)
    mask_row = maskf.reshape(1, batch)
    mask_col = maskf.reshape(batch, 1)

    block_rows = _BLOCK_ROWS
    grid = num_pids // block_rows

    out = pl.pallas_call(
        _oim_tc_body,
        grid=(grid,),
        in_specs=[
            pl.BlockSpec((block_rows, feat), lambda j: (j, 0)),
            pl.BlockSpec((batch, feat), lambda j: (0, 0)),
            pl.BlockSpec((1, batch), lambda j: (0, 0)),
            pl.BlockSpec((batch, 1), lambda j: (0, 0)),
            pl.BlockSpec((batch, feat), lambda j: (0, 0)),
        ],
        out_specs=pl.BlockSpec(memory_space=pltpu.SMEM),
        out_shape=jax.ShapeDtypeStruct((1, 1), jnp.float32),
        scratch_shapes=[
            pltpu.VMEM((_CHUNKS * _ACCW, batch), jnp.float32),
            pltpu.VMEM((_CHUNKS * _ACCW, batch), jnp.float32),
        ],
        compiler_params=pltpu.CompilerParams(
            dimension_semantics=("arbitrary",),
        ),
    )(lut, xs, mask_row, mask_col, g)

    return out[0, 0]
